# baseline (speedup 1.0000x reference)
import jax
import jax.numpy as jnp
from jax.experimental import pallas as pl
from jax.experimental.pallas import tpu as pltpu

_P, _N, _NFEAT, _NHID, _SHID = 3, 4096, 128, 64, 32
_BM = 512
_NM = _N // _BM


def _fused_body(adjl_ref, adjr_ref, s1_ref, s2_ref, wg_ref, b_ref, a_ref,
                msk_ref, sb1_ref, sb2_ref, l1_ref, l2_ref,
                wa_ref, ba_ref, qa_ref, wdt_ref, bd_ref,
                hh1_ref, loss_ref,
                f1_s, f2_s, h1_s, h2_s, t1_s, t2_s):
    i = pl.program_id(0)
    m = pl.program_id(1)

    @pl.when((i == 0) & (m == 0))
    def _init():
        for j in range(_P):
            wj = wg_ref[j]
            f1_s[j] = jnp.dot(s1_ref[0], wj, preferred_element_type=jnp.float32)
            f2_s[j] = jnp.dot(s2_ref[0], wj, preferred_element_type=jnp.float32)
        t1_s[...] = jnp.zeros_like(t1_s)
        t2_s[...] = jnp.zeros_like(t2_s)

    b = b_ref[0]
    a = a_ref[0]
    wa = wa_ref[...]
    ba = ba_ref[...]
    f1 = f1_s[i]
    f2 = f2_s[i]
    hb = _BM // 2
    for half, adj in enumerate((adjl_ref[0, 0], adjr_ref[0, 0])):
        y1 = jnp.dot(adj, f1, preferred_element_type=jnp.float32) + b
        y2 = jnp.dot(adj, f2, preferred_element_type=jnp.float32) + b
        h1 = jnp.where(y1 >= 0, y1, a * y1)
        h2 = jnp.where(y2 >= 0, y2, a * y2)
        hh1_ref[0, pl.ds(half * hb, hb), :] = h1
        h1_s[i, pl.ds(m * _BM + half * hb, hb), :] = h1
        h2_s[i, pl.ds(m * _BM + half * hb, hb), :] = h2
        u1 = jnp.tanh(jnp.dot(h1, wa, preferred_element_type=jnp.float32) + ba)
        u2 = jnp.tanh(jnp.dot(h2, wa, preferred_element_type=jnp.float32) + ba)
        t1_s[i] += jnp.sum(u1, axis=0, keepdims=True)
        t2_s[i] += jnp.sum(u2, axis=0, keepdims=True)

    @pl.when((i == _P - 1) & (m == _NM - 1))
    def _tail():
        qa = qa_ref[...]
        w1s = [jnp.sum(t1_s[j] * qa) / _N for j in range(_P)]
        w2s = [jnp.sum(t2_s[j] * qa) / _N for j in range(_P)]

        def _softmax3(ws):
            mx = jnp.maximum(jnp.maximum(ws[0], ws[1]), ws[2])
            es = [jnp.exp(w - mx) for w in ws]
            s = es[0] + es[1] + es[2]
            return [e / s for e in es]

        b1 = _softmax3(w1s)
        b2 = _softmax3(w2s)
        ha1 = b1[0] * h1_s[0] + b1[1] * h1_s[1] + b1[2] * h1_s[2]
        ha2 = b2[0] * h2_s[0] + b2[1] * h2_s[1] + b2[2] * h2_s[2]

        msk = msk_ref[...]
        r = jnp.dot(msk, ha1, preferred_element_type=jnp.float32)
        c = jax.nn.sigmoid(r / jnp.sum(msk))
        u = jnp.dot(c, wdt_ref[...], preferred_element_type=jnp.float32)
        bd = bd_ref[0, 0]
        sc1 = jnp.sum(ha1 * u, axis=1, keepdims=True) + bd + sb1_ref[...]
        sc2 = jnp.sum(ha2 * u, axis=1, keepdims=True) + bd + sb2_ref[...]

        def _bce(x, t):
            return jnp.maximum(x, 0.0) - x * t + jnp.log1p(jnp.exp(-jnp.abs(x)))

        loss = (jnp.sum(_bce(sc1, l1_ref[...]), keepdims=True)
                + jnp.sum(_bce(sc2, l2_ref[...]), keepdims=True))
        loss_ref[...] = loss / (2 * _N)


def kernel(seq1, seq2, lbl, adjs, sparse, msk, samp_bias1, samp_bias2,
           W_gcn, b_gcn, a_prelu, W_att, b_att, q_att, W_disc, b_disc):
    del sparse
    b3 = b_gcn.reshape(_P, 1, _NHID)
    a3 = jnp.broadcast_to(a_prelu[:, None, None], (_P, 1, _NHID))
    const = lambda i, m: (0, 0)
    const3 = lambda i, m: (0, 0, 0)
    per_i = lambda i, m: (i, 0, 0)
    hh1, loss = pl.pallas_call(
        _fused_body,
        grid=(_P, _NM),
        in_specs=[
            pl.BlockSpec((1, 1, _BM // 2, _N), lambda i, m: (i, 0, 2 * m, 0)),
            pl.BlockSpec((1, 1, _BM // 2, _N), lambda i, m: (i, 0, 2 * m + 1, 0)),
            pl.BlockSpec((1, _N, _NFEAT), const3),
            pl.BlockSpec((1, _N, _NFEAT), const3),
            pl.BlockSpec((_P, _NFEAT, _NHID), const3),
            pl.BlockSpec((1, 1, _NHID), per_i),
            pl.BlockSpec((1, 1, _NHID), per_i),
            pl.BlockSpec((1, _N), const),
            pl.BlockSpec((_N, 1), const),
            pl.BlockSpec((_N, 1), const),
            pl.BlockSpec((_N, 1), const),
            pl.BlockSpec((_N, 1), const),
            pl.BlockSpec((_NHID, _SHID), const),
            pl.BlockSpec((1, _SHID), const),
            pl.BlockSpec((1, _SHID), const),
            pl.BlockSpec((_NHID, _NHID), const),
            pl.BlockSpec((1, 1), const),
        ],
        out_specs=[
            pl.BlockSpec((1, _BM, _NHID), lambda i, m: (i, m, 0)),
            pl.BlockSpec((1, 1), const),
        ],
        out_shape=[
            jax.ShapeDtypeStruct((_P, _N, _NHID), jnp.float32),
            jax.ShapeDtypeStruct((1, 1), jnp.float32),
        ],
        scratch_shapes=[
            pltpu.VMEM((_P, _N, _NHID), jnp.float32),
            pltpu.VMEM((_P, _N, _NHID), jnp.float32),
            pltpu.VMEM((_P, _N, _NHID), jnp.float32),
            pltpu.VMEM((_P, _N, _NHID), jnp.float32),
            pltpu.VMEM((_P, 1, _SHID), jnp.float32),
            pltpu.VMEM((_P, 1, _SHID), jnp.float32),
        ],
    )(adjs, adjs, seq1, seq2, W_gcn, b3, a3,
      msk,
      samp_bias1.reshape(_N, 1), samp_bias2.reshape(_N, 1),
      lbl[:, :_N].reshape(_N, 1), lbl[:, _N:].reshape(_N, 1),
      W_att, b_att.reshape(1, _SHID), q_att.reshape(1, _SHID),
      W_disc.T, b_disc.reshape(1, 1))

    return (loss[0, 0], hh1)


# speedup vs baseline: 1.0237x; 1.0237x over previous
import jax
import jax.numpy as jnp
from jax.experimental import pallas as pl
from jax.experimental.pallas import tpu as pltpu

_P, _N, _NFEAT, _NHID, _SHID = 3, 4096, 128, 64, 32
_BM = 1024
_NM = _N // _BM


def _fused_body(adjl_ref, adjr_ref, s1_ref, s2_ref, wg_ref, b_ref, a_ref,
                msk_ref, sb1_ref, sb2_ref, l1_ref, l2_ref,
                wa_ref, ba_ref, qa_ref, wdt_ref, bd_ref,
                hh1_ref, loss_ref,
                f1_s, f2_s, h1_s, h2_s, t1_s, t2_s):
    i = pl.program_id(0)
    m = pl.program_id(1)

    @pl.when(m == 0)
    def _init():
        wj = wg_ref[0]
        f1_s[...] = jnp.dot(
            s1_ref[0], wj,
            preferred_element_type=jnp.float32).astype(jnp.bfloat16)
        f2_s[...] = jnp.dot(
            s2_ref[0], wj,
            preferred_element_type=jnp.float32).astype(jnp.bfloat16)

    @pl.when((i == 0) & (m == 0))
    def _zero():
        t1_s[...] = jnp.zeros_like(t1_s)
        t2_s[...] = jnp.zeros_like(t2_s)

    adjl = adjl_ref[0, 0]
    adjr = adjr_ref[0, 0]
    b = b_ref[0]
    a = a_ref[0]
    wa = wa_ref[...]
    ba = ba_ref[...]
    nh = _N // 2
    f1t = f1_s[:nh, :].astype(jnp.float32)
    f1b = f1_s[nh:, :].astype(jnp.float32)
    f2t = f2_s[:nh, :].astype(jnp.float32)
    f2b = f2_s[nh:, :].astype(jnp.float32)
    y1 = (jnp.dot(adjl, f1t, preferred_element_type=jnp.float32)
          + jnp.dot(adjr, f1b, preferred_element_type=jnp.float32) + b)
    y2 = (jnp.dot(adjl, f2t, preferred_element_type=jnp.float32)
          + jnp.dot(adjr, f2b, preferred_element_type=jnp.float32) + b)
    h1 = jnp.where(y1 >= 0, y1, a * y1)
    h2 = jnp.where(y2 >= 0, y2, a * y2)
    hh1_ref[0] = h1
    h1_s[i, pl.ds(m * _BM, _BM), :] = h1.astype(jnp.bfloat16)
    h2_s[i, pl.ds(m * _BM, _BM), :] = h2.astype(jnp.bfloat16)
    u1 = jnp.tanh(jnp.dot(h1, wa, preferred_element_type=jnp.float32) + ba)
    u2 = jnp.tanh(jnp.dot(h2, wa, preferred_element_type=jnp.float32) + ba)
    t1_s[i] += jnp.sum(u1, axis=0, keepdims=True)
    t2_s[i] += jnp.sum(u2, axis=0, keepdims=True)

    @pl.when((i == _P - 1) & (m == _NM - 1))
    def _tail():
        qa = qa_ref[...]
        w1s = [jnp.sum(t1_s[j] * qa) / _N for j in range(_P)]
        w2s = [jnp.sum(t2_s[j] * qa) / _N for j in range(_P)]

        def _softmax3(ws):
            mx = jnp.maximum(jnp.maximum(ws[0], ws[1]), ws[2])
            es = [jnp.exp(w - mx) for w in ws]
            s = es[0] + es[1] + es[2]
            return [e / s for e in es]

        b1 = _softmax3(w1s)
        b2 = _softmax3(w2s)

        def _bce(x, t):
            return jnp.maximum(x, 0.0) - x * t + jnp.log1p(jnp.exp(-jnp.abs(x)))

        bd = bd_ref[0, 0]
        msk = msk_ref[...]

        ha1 = (b1[0] * h1_s[0].astype(jnp.float32)
               + b1[1] * h1_s[1].astype(jnp.float32)
               + b1[2] * h1_s[2].astype(jnp.float32))
        r = jnp.dot(msk, ha1, preferred_element_type=jnp.float32)
        c = jax.nn.sigmoid(r / jnp.sum(msk))
        u = jnp.dot(c, wdt_ref[...], preferred_element_type=jnp.float32)
        sc1 = jnp.sum(ha1 * u, axis=1, keepdims=True) + bd + sb1_ref[...]
        loss1 = jnp.sum(_bce(sc1, l1_ref[...]), keepdims=True)

        ha2 = (b2[0] * h2_s[0].astype(jnp.float32)
               + b2[1] * h2_s[1].astype(jnp.float32)
               + b2[2] * h2_s[2].astype(jnp.float32))
        sc2 = jnp.sum(ha2 * u, axis=1, keepdims=True) + bd + sb2_ref[...]
        loss2 = jnp.sum(_bce(sc2, l2_ref[...]), keepdims=True)
        loss_ref[...] = (loss1 + loss2) / (2 * _N)


def kernel(seq1, seq2, lbl, adjs, sparse, msk, samp_bias1, samp_bias2,
           W_gcn, b_gcn, a_prelu, W_att, b_att, q_att, W_disc, b_disc):
    del sparse
    b3 = b_gcn.reshape(_P, 1, _NHID)
    a3 = jnp.broadcast_to(a_prelu[:, None, None], (_P, 1, _NHID))
    const = lambda i, m: (0, 0)
    const3 = lambda i, m: (0, 0, 0)
    per_i = lambda i, m: (i, 0, 0)
    hh1, loss = pl.pallas_call(
        _fused_body,
        grid=(_P, _NM),
        in_specs=[
            pl.BlockSpec((1, 1, _BM, _N // 2), lambda i, m: (i, 0, m, 0)),
            pl.BlockSpec((1, 1, _BM, _N // 2), lambda i, m: (i, 0, m, 1)),
            pl.BlockSpec((1, _N, _NFEAT), const3),
            pl.BlockSpec((1, _N, _NFEAT), const3),
            pl.BlockSpec((1, _NFEAT, _NHID), per_i),
            pl.BlockSpec((1, 1, _NHID), per_i),
            pl.BlockSpec((1, 1, _NHID), per_i),
            pl.BlockSpec((1, _N), const),
            pl.BlockSpec((_N, 1), const),
            pl.BlockSpec((_N, 1), const),
            pl.BlockSpec((_N, 1), const),
            pl.BlockSpec((_N, 1), const),
            pl.BlockSpec((_NHID, _SHID), const),
            pl.BlockSpec((1, _SHID), const),
            pl.BlockSpec((1, _SHID), const),
            pl.BlockSpec((_NHID, _NHID), const),
            pl.BlockSpec((1, 1), const),
        ],
        out_specs=[
            pl.BlockSpec((1, _BM, _NHID), lambda i, m: (i, m, 0)),
            pl.BlockSpec((1, 1), const),
        ],
        out_shape=[
            jax.ShapeDtypeStruct((_P, _N, _NHID), jnp.float32),
            jax.ShapeDtypeStruct((1, 1), jnp.float32),
        ],
        scratch_shapes=[
            pltpu.VMEM((_N, _NHID), jnp.bfloat16),
            pltpu.VMEM((_N, _NHID), jnp.bfloat16),
            pltpu.VMEM((_P, _N, _NHID), jnp.bfloat16),
            pltpu.VMEM((_P, _N, _NHID), jnp.bfloat16),
            pltpu.VMEM((_P, 1, _SHID), jnp.float32),
            pltpu.VMEM((_P, 1, _SHID), jnp.float32),
        ],
    )(adjs, adjs, seq1, seq2, W_gcn, b3, a3,
      msk,
      samp_bias1.reshape(_N, 1), samp_bias2.reshape(_N, 1),
      lbl[:, :_N].reshape(_N, 1), lbl[:, _N:].reshape(_N, 1),
      W_att, b_att.reshape(1, _SHID), q_att.reshape(1, _SHID),
      W_disc.T, b_disc.reshape(1, 1))

    return (loss[0, 0], hh1)


# speedup vs baseline: 1.0263x; 1.0025x over previous
import jax
import jax.numpy as jnp
from jax.experimental import pallas as pl
from jax.experimental.pallas import tpu as pltpu

_P, _N, _NFEAT, _NHID, _SHID = 3, 4096, 128, 64, 32
_BM = 1024
_NM = _N // _BM


def _fused_body(adj0_ref, adj1_ref, adj2_ref, adj3_ref,
                s1_ref, s2_ref, wg_ref, b_ref, a_ref,
                msk_ref, sb1_ref, sb2_ref, l1_ref, l2_ref,
                wa_ref, ba_ref, qa_ref, wdt_ref, bd_ref,
                hh1_ref, loss_ref,
                f1_s, f2_s, h1_s, h2_s, t1_s, t2_s):
    i = pl.program_id(0)
    m = pl.program_id(1)

    @pl.when(m == 0)
    def _init():
        wj = wg_ref[0]
        f1_s[...] = jnp.dot(
            s1_ref[0], wj,
            preferred_element_type=jnp.float32).astype(jnp.bfloat16)
        f2_s[...] = jnp.dot(
            s2_ref[0], wj,
            preferred_element_type=jnp.float32).astype(jnp.bfloat16)

    @pl.when((i == 0) & (m == 0))
    def _zero():
        t1_s[...] = jnp.zeros_like(t1_s)
        t2_s[...] = jnp.zeros_like(t2_s)

    b = b_ref[0]
    a = a_ref[0]
    wa = wa_ref[...]
    ba = ba_ref[...]
    nq = _N // 4
    y1 = b.astype(jnp.float32)
    y2 = b.astype(jnp.float32)
    for q, aref in enumerate((adj0_ref, adj1_ref, adj2_ref, adj3_ref)):
        adj = aref[0, 0]
        fq1 = f1_s[q * nq:(q + 1) * nq, :].astype(jnp.float32)
        fq2 = f2_s[q * nq:(q + 1) * nq, :].astype(jnp.float32)
        y1 = y1 + jnp.dot(adj, fq1, preferred_element_type=jnp.float32)
        y2 = y2 + jnp.dot(adj, fq2, preferred_element_type=jnp.float32)
    h1 = jnp.where(y1 >= 0, y1, a * y1)
    h2 = jnp.where(y2 >= 0, y2, a * y2)
    hh1_ref[0] = h1
    h1_s[i, pl.ds(m * _BM, _BM), :] = h1.astype(jnp.bfloat16)
    h2_s[i, pl.ds(m * _BM, _BM), :] = h2.astype(jnp.bfloat16)
    u1 = jnp.tanh(jnp.dot(h1, wa, preferred_element_type=jnp.float32) + ba)
    u2 = jnp.tanh(jnp.dot(h2, wa, preferred_element_type=jnp.float32) + ba)
    t1_s[i] += jnp.sum(u1, axis=0, keepdims=True)
    t2_s[i] += jnp.sum(u2, axis=0, keepdims=True)

    @pl.when((i == _P - 1) & (m == _NM - 1))
    def _tail():
        qa = qa_ref[...]
        w1s = [jnp.sum(t1_s[j] * qa) / _N for j in range(_P)]
        w2s = [jnp.sum(t2_s[j] * qa) / _N for j in range(_P)]

        def _softmax3(ws):
            mx = jnp.maximum(jnp.maximum(ws[0], ws[1]), ws[2])
            es = [jnp.exp(w - mx) for w in ws]
            s = es[0] + es[1] + es[2]
            return [e / s for e in es]

        b1 = _softmax3(w1s)
        b2 = _softmax3(w2s)

        def _bce(x, t):
            return jnp.maximum(x, 0.0) - x * t + jnp.log1p(jnp.exp(-jnp.abs(x)))

        bd = bd_ref[0, 0]
        msk = msk_ref[...]

        ha1 = (b1[0] * h1_s[0].astype(jnp.float32)
               + b1[1] * h1_s[1].astype(jnp.float32)
               + b1[2] * h1_s[2].astype(jnp.float32))
        r = jnp.dot(msk, ha1, preferred_element_type=jnp.float32)
        c = jax.nn.sigmoid(r / jnp.sum(msk))
        u = jnp.dot(c, wdt_ref[...], preferred_element_type=jnp.float32)
        sc1 = jnp.sum(ha1 * u, axis=1, keepdims=True) + bd + sb1_ref[...]
        loss1 = jnp.sum(_bce(sc1, l1_ref[...]), keepdims=True)

        ha2 = (b2[0] * h2_s[0].astype(jnp.float32)
               + b2[1] * h2_s[1].astype(jnp.float32)
               + b2[2] * h2_s[2].astype(jnp.float32))
        sc2 = jnp.sum(ha2 * u, axis=1, keepdims=True) + bd + sb2_ref[...]
        loss2 = jnp.sum(_bce(sc2, l2_ref[...]), keepdims=True)
        loss_ref[...] = (loss1 + loss2) / (2 * _N)


def kernel(seq1, seq2, lbl, adjs, sparse, msk, samp_bias1, samp_bias2,
           W_gcn, b_gcn, a_prelu, W_att, b_att, q_att, W_disc, b_disc):
    del sparse
    b3 = b_gcn.reshape(_P, 1, _NHID)
    a3 = jnp.broadcast_to(a_prelu[:, None, None], (_P, 1, _NHID))
    const = lambda i, m: (0, 0)
    const3 = lambda i, m: (0, 0, 0)
    per_i = lambda i, m: (i, 0, 0)
    hh1, loss = pl.pallas_call(
        _fused_body,
        grid=(_P, _NM),
        in_specs=[
            pl.BlockSpec((1, 1, _BM, _N // 4), lambda i, m: (i, 0, m, 0)),
            pl.BlockSpec((1, 1, _BM, _N // 4), lambda i, m: (i, 0, m, 1)),
            pl.BlockSpec((1, 1, _BM, _N // 4), lambda i, m: (i, 0, m, 2)),
            pl.BlockSpec((1, 1, _BM, _N // 4), lambda i, m: (i, 0, m, 3)),
            pl.BlockSpec((1, _N, _NFEAT), const3),
            pl.BlockSpec((1, _N, _NFEAT), const3),
            pl.BlockSpec((1, _NFEAT, _NHID), per_i),
            pl.BlockSpec((1, 1, _NHID), per_i),
            pl.BlockSpec((1, 1, _NHID), per_i),
            pl.BlockSpec((1, _N), const),
            pl.BlockSpec((_N, 1), const),
            pl.BlockSpec((_N, 1), const),
            pl.BlockSpec((_N, 1), const),
            pl.BlockSpec((_N, 1), const),
            pl.BlockSpec((_NHID, _SHID), const),
            pl.BlockSpec((1, _SHID), const),
            pl.BlockSpec((1, _SHID), const),
            pl.BlockSpec((_NHID, _NHID), const),
            pl.BlockSpec((1, 1), const),
        ],
        out_specs=[
            pl.BlockSpec((1, _BM, _NHID), lambda i, m: (i, m, 0)),
            pl.BlockSpec((1, 1), const),
        ],
        out_shape=[
            jax.ShapeDtypeStruct((_P, _N, _NHID), jnp.float32),
            jax.ShapeDtypeStruct((1, 1), jnp.float32),
        ],
        scratch_shapes=[
            pltpu.VMEM((_N, _NHID), jnp.bfloat16),
            pltpu.VMEM((_N, _NHID), jnp.bfloat16),
            pltpu.VMEM((_P, _N, _NHID), jnp.bfloat16),
            pltpu.VMEM((_P, _N, _NHID), jnp.bfloat16),
            pltpu.VMEM((_P, 1, _SHID), jnp.float32),
            pltpu.VMEM((_P, 1, _SHID), jnp.float32),
        ],
    )(adjs, adjs, adjs, adjs, seq1, seq2, W_gcn, b3, a3,
      msk,
      samp_bias1.reshape(_N, 1), samp_bias2.reshape(_N, 1),
      lbl[:, :_N].reshape(_N, 1), lbl[:, _N:].reshape(_N, 1),
      W_att, b_att.reshape(1, _SHID), q_att.reshape(1, _SHID),
      W_disc.T, b_disc.reshape(1, 1))

    return (loss[0, 0], hh1)


# speedup vs baseline: 1.1218x; 1.0931x over previous
import jax
import jax.numpy as jnp
from jax.experimental import pallas as pl
from jax.experimental.pallas import tpu as pltpu

_P, _N, _NFEAT, _NHID, _SHID = 3, 4096, 128, 64, 32
_BM = 1024
_NM = _N // _BM


def _fused_body(adj0_ref, adj1_ref, adj2_ref, adj3_ref,
                s1_ref, s2_ref, wg_ref, b_ref, a_ref,
                msk_ref, sb1_ref, sb2_ref, l1_ref, l2_ref,
                wa_ref, ba_ref, qa_ref, wdt_ref, bd_ref,
                hh1_ref, loss_ref,
                f1_s, f2_s, h1_s, h2_s, t1_s, t2_s):
    i = pl.program_id(0)
    m = pl.program_id(1)

    @pl.when(m == 0)
    def _init():
        wj = wg_ref[0]
        f1_s[...] = jnp.dot(
            s1_ref[0], wj,
            preferred_element_type=jnp.float32).astype(jnp.bfloat16)
        f2_s[...] = jnp.dot(
            s2_ref[0], wj,
            preferred_element_type=jnp.float32).astype(jnp.bfloat16)

    @pl.when((i == 0) & (m == 0))
    def _zero():
        t1_s[...] = jnp.zeros_like(t1_s)
        t2_s[...] = jnp.zeros_like(t2_s)

    b = b_ref[0]
    a = a_ref[0]
    wa = wa_ref[...]
    ba = ba_ref[...]
    nq = _N // 4
    y1 = b.astype(jnp.float32)
    y2 = b.astype(jnp.float32)
    for q, aref in enumerate((adj0_ref, adj1_ref, adj2_ref, adj3_ref)):
        adj = aref[0, 0]
        fq1 = f1_s[q * nq:(q + 1) * nq, :].astype(jnp.float32)
        fq2 = f2_s[q * nq:(q + 1) * nq, :].astype(jnp.float32)
        y1 = y1 + jnp.dot(adj, fq1, preferred_element_type=jnp.float32)
        y2 = y2 + jnp.dot(adj, fq2, preferred_element_type=jnp.float32)
    h1 = jnp.where(y1 >= 0, y1, a * y1)
    h2 = jnp.where(y2 >= 0, y2, a * y2)
    hh1_ref[0] = h1
    h1_s[i, pl.ds(m * _BM, _BM), :] = h1.astype(jnp.bfloat16)
    h2_s[i, pl.ds(m * _BM, _BM), :] = h2.astype(jnp.bfloat16)
    u1 = jnp.tanh(jnp.dot(h1, wa, preferred_element_type=jnp.float32) + ba)
    u2 = jnp.tanh(jnp.dot(h2, wa, preferred_element_type=jnp.float32) + ba)
    t1_s[i] += jnp.sum(u1, axis=0, keepdims=True)
    t2_s[i] += jnp.sum(u2, axis=0, keepdims=True)

    @pl.when((i == _P - 1) & (m == _NM - 1))
    def _tail():
        qa = qa_ref[...]
        w1s = [jnp.sum(t1_s[j] * qa) / _N for j in range(_P)]
        w2s = [jnp.sum(t2_s[j] * qa) / _N for j in range(_P)]

        def _softmax3(ws):
            mx = jnp.maximum(jnp.maximum(ws[0], ws[1]), ws[2])
            es = [jnp.exp(w - mx) for w in ws]
            s = es[0] + es[1] + es[2]
            return [e / s for e in es]

        b1 = _softmax3(w1s)
        b2 = _softmax3(w2s)

        def _bce(x, t):
            return jnp.maximum(x, 0.0) - x * t + jnp.log1p(jnp.exp(-jnp.abs(x)))

        bd = bd_ref[0, 0]
        msk = msk_ref[...]
        mskb = msk.astype(jnp.bfloat16)

        r = (b1[0] * jnp.dot(mskb, h1_s[0], preferred_element_type=jnp.float32)
             + b1[1] * jnp.dot(mskb, h1_s[1], preferred_element_type=jnp.float32)
             + b1[2] * jnp.dot(mskb, h1_s[2], preferred_element_type=jnp.float32))
        c = jax.nn.sigmoid(r / jnp.sum(msk))
        u = jnp.dot(c, wdt_ref[...], preferred_element_type=jnp.float32)
        ub = jnp.swapaxes(u, 0, 1).astype(jnp.bfloat16)

        sc1 = (b1[0] * jnp.dot(h1_s[0], ub, preferred_element_type=jnp.float32)
               + b1[1] * jnp.dot(h1_s[1], ub, preferred_element_type=jnp.float32)
               + b1[2] * jnp.dot(h1_s[2], ub, preferred_element_type=jnp.float32))
        sc2 = (b2[0] * jnp.dot(h2_s[0], ub, preferred_element_type=jnp.float32)
               + b2[1] * jnp.dot(h2_s[1], ub, preferred_element_type=jnp.float32)
               + b2[2] * jnp.dot(h2_s[2], ub, preferred_element_type=jnp.float32))
        sc1 = sc1.reshape(_N // 128, 128) + bd + sb1_ref[...]
        sc2 = sc2.reshape(_N // 128, 128) + bd + sb2_ref[...]
        loss = (jnp.sum(_bce(sc1, l1_ref[...]), keepdims=True)
                + jnp.sum(_bce(sc2, l2_ref[...]), keepdims=True))
        loss_ref[...] = loss / (2 * _N)


def kernel(seq1, seq2, lbl, adjs, sparse, msk, samp_bias1, samp_bias2,
           W_gcn, b_gcn, a_prelu, W_att, b_att, q_att, W_disc, b_disc):
    del sparse
    b3 = b_gcn.reshape(_P, 1, _NHID)
    a3 = jnp.broadcast_to(a_prelu[:, None, None], (_P, 1, _NHID))
    const = lambda i, m: (0, 0)
    const3 = lambda i, m: (0, 0, 0)
    per_i = lambda i, m: (i, 0, 0)
    hh1, loss = pl.pallas_call(
        _fused_body,
        grid=(_P, _NM),
        in_specs=[
            pl.BlockSpec((1, 1, _BM, _N // 4), lambda i, m: (i, 0, m, 0)),
            pl.BlockSpec((1, 1, _BM, _N // 4), lambda i, m: (i, 0, m, 1)),
            pl.BlockSpec((1, 1, _BM, _N // 4), lambda i, m: (i, 0, m, 2)),
            pl.BlockSpec((1, 1, _BM, _N // 4), lambda i, m: (i, 0, m, 3)),
            pl.BlockSpec((1, _N, _NFEAT), const3),
            pl.BlockSpec((1, _N, _NFEAT), const3),
            pl.BlockSpec((1, _NFEAT, _NHID), per_i),
            pl.BlockSpec((1, 1, _NHID), per_i),
            pl.BlockSpec((1, 1, _NHID), per_i),
            pl.BlockSpec((1, _N), const),
            pl.BlockSpec((_N // 128, 128), const),
            pl.BlockSpec((_N // 128, 128), const),
            pl.BlockSpec((_N // 128, 128), const),
            pl.BlockSpec((_N // 128, 128), const),
            pl.BlockSpec((_NHID, _SHID), const),
            pl.BlockSpec((1, _SHID), const),
            pl.BlockSpec((1, _SHID), const),
            pl.BlockSpec((_NHID, _NHID), const),
            pl.BlockSpec((1, 1), const),
        ],
        out_specs=[
            pl.BlockSpec((1, _BM, _NHID), lambda i, m: (i, m, 0)),
            pl.BlockSpec((1, 1), const),
        ],
        out_shape=[
            jax.ShapeDtypeStruct((_P, _N, _NHID), jnp.float32),
            jax.ShapeDtypeStruct((1, 1), jnp.float32),
        ],
        scratch_shapes=[
            pltpu.VMEM((_N, _NHID), jnp.bfloat16),
            pltpu.VMEM((_N, _NHID), jnp.bfloat16),
            pltpu.VMEM((_P, _N, _NHID), jnp.bfloat16),
            pltpu.VMEM((_P, _N, _NHID), jnp.bfloat16),
            pltpu.VMEM((_P, 1, _SHID), jnp.float32),
            pltpu.VMEM((_P, 1, _SHID), jnp.float32),
        ],
    )(adjs, adjs, adjs, adjs, seq1, seq2, W_gcn, b3, a3,
      msk,
      samp_bias1.reshape(_N // 128, 128), samp_bias2.reshape(_N // 128, 128),
      lbl[:, :_N].reshape(_N // 128, 128), lbl[:, _N:].reshape(_N // 128, 128),
      W_att, b_att.reshape(1, _SHID), q_att.reshape(1, _SHID),
      W_disc.T, b_disc.reshape(1, 1))

    return (loss[0, 0], hh1)
